# baseline (speedup 1.0000x reference)
import functools

import jax
import jax.numpy as jnp
from jax import lax
from jax.experimental import pallas as pl
from jax.experimental.pallas import tpu as pltpu
from jax.experimental.pallas import tpu_sc as plsc

N = 10000
E = 320000
D_IN = 128
H = 64
F32 = jnp.float32

NC = 2
NS = 16
NW = NC * NS
NP = 10240
NPT = NP // NS
EP = 327680
ER = EP // 128
RW = ER // NW

K_AGG = 4
K_CLS = 2


def _mesh():
    return plsc.VectorSubcoreMesh(
        core_axis_name="c", subcore_axis_name="s", num_cores=NC, num_subcores=NS
    )


def _worker_id():
    return lax.axis_index("s") * NC + lax.axis_index("c"), lax.axis_index("c"), \
        lax.axis_index("s")


RW_FAST = 120
RW_SLOW = 40
FAST_CID = 0


def _edge_rows(cid, sid, blk, k):
    rw = jnp.where(cid == FAST_CID, RW_FAST, RW_SLOW)
    base = jnp.where(cid == FAST_CID, sid * RW_FAST,
                     NS * RW_FAST + sid * RW_SLOW)
    return base + blk * k, rw // k


DW = 16


@functools.partial(
    pl.kernel,
    out_type=jax.ShapeDtypeStruct((NC, NP, DW), F32),
    mesh=_mesh(),
    compiler_params=pltpu.CompilerParams(use_tc_tiling_on_sc=False),
    scratch_types=[
        pltpu.VMEM((K_AGG, 128), jnp.int32),
        pltpu.VMEM((128, DW), F32),
        pltpu.VMEM((NPT, DW), F32),
        pltpu.VMEM_SHARED((NP, DW), F32),
    ],
)
def _deg_kernel(dst_hbm, ones_hbm, zeros_hbm, out_hbm, idx_v, ones_v, stage_v,
                acc_sh):
    wid, cid, sid = _worker_id()
    pltpu.sync_copy(zeros_hbm, stage_v)
    pltpu.sync_copy(stage_v, acc_sh.at[pl.ds(sid * NPT, NPT)])
    pltpu.sync_copy(ones_hbm, ones_v)
    plsc.subcore_barrier()

    def body(b, carry):
        rowbase = wid * RW + b * K_AGG
        pltpu.sync_copy(dst_hbm.at[pl.ds(rowbase, K_AGG)], idx_v)
        for j in range(K_AGG):
            pltpu.sync_copy(ones_v, acc_sh.at[idx_v.at[j]], add=True)
        return carry

    lax.fori_loop(0, RW // K_AGG, body, 0)
    plsc.subcore_barrier()
    pltpu.sync_copy(acc_sh.at[pl.ds(sid * NPT, NPT)], stage_v)
    pltpu.sync_copy(stage_v, out_hbm.at[cid, pl.ds(sid * NPT, NPT)])


@functools.partial(
    pl.kernel,
    out_type=jax.ShapeDtypeStruct((NC, NP, H), F32),
    mesh=_mesh(),
    compiler_params=pltpu.CompilerParams(use_tc_tiling_on_sc=False),
    scratch_types=[
        pltpu.VMEM((K_AGG, 128), jnp.int32),
        pltpu.VMEM((K_AGG, 128), jnp.int32),
        pltpu.VMEM((K_AGG, 128), jnp.int32),
        pltpu.VMEM((K_AGG, 128), jnp.int32),
        pltpu.VMEM((K_AGG * 128, H), F32),
        pltpu.VMEM((K_AGG * 128, H), F32),
        pltpu.SemaphoreType.DMA,
        pltpu.SemaphoreType.DMA,
        pltpu.VMEM_SHARED((NP, H), F32),
    ],
)
def _agg_kernel(q_hbm, src_hbm, dst_hbm, zeros_hbm, out_hbm, sidx0, sidx1,
                didx0, didx1, rows0, rows1, sem0, sem1, acc_sh):
    wid, cid, sid = _worker_id()
    sidx = (sidx0, sidx1)
    didx = (didx0, didx1)
    rows = (rows0, rows1)
    sems = (sem0, sem1)
    _r0 = K_AGG * 128
    _r1 = NPT - _r0
    pltpu.sync_copy(zeros_hbm.at[pl.ds(0, _r0)], rows0)
    pltpu.sync_copy(zeros_hbm.at[pl.ds(_r0, _r1)], rows1.at[pl.ds(0, _r1)])
    pltpu.sync_copy(rows0, acc_sh.at[pl.ds(sid * NPT, _r0)])
    pltpu.sync_copy(rows1.at[pl.ds(0, _r1)],
                    acc_sh.at[pl.ds(sid * NPT + _r0, _r1)])
    plsc.subcore_barrier()
    _, nblk = _edge_rows(cid, sid, 0, K_AGG)

    def fire(b, s):
        rowbase, _ = _edge_rows(cid, sid, b, K_AGG)
        pltpu.sync_copy(src_hbm.at[pl.ds(rowbase, K_AGG)], sidx[s])
        pltpu.sync_copy(dst_hbm.at[pl.ds(rowbase, K_AGG)], didx[s])
        return [
            pltpu.async_copy(q_hbm.at[sidx[s].at[j]],
                             rows[s].at[pl.ds(j * 128, 128)], sems[s])
            for j in range(K_AGG)
        ]

    def drain_consume(descs, s):
        for d in descs:
            d.wait()
        for j in range(K_AGG):
            pltpu.sync_copy(rows[s].at[pl.ds(j * 128, 128)],
                            acc_sh.at[didx[s].at[j]], add=True)

    d0 = fire(0, 0)

    def body(b2, carry):
        b = b2 * 2
        d1 = fire(b + 1, 1)
        drain_consume(d0, 0)

        @pl.when(b + 2 < nblk)
        def _():
            fire(b + 2, 0)

        drain_consume(d1, 1)
        return carry

    lax.fori_loop(0, nblk // 2, body, 0)
    plsc.subcore_barrier()
    pltpu.sync_copy(acc_sh.at[pl.ds(sid * NPT, _r0)], rows0)
    pltpu.sync_copy(acc_sh.at[pl.ds(sid * NPT + _r0, _r1)],
                    rows1.at[pl.ds(0, _r1)])
    pltpu.sync_copy(rows0, out_hbm.at[cid, pl.ds(sid * NPT, _r0)])
    pltpu.sync_copy(rows1.at[pl.ds(0, _r1)],
                    out_hbm.at[cid, pl.ds(sid * NPT + _r0, _r1)])


def _edge_gather_body(a_hbm, b_hbm, src_hbm, dst_hbm, o_hbm, sidx0, sidx1,
                      didx0, didx1, bufa0, bufa1, bufb0, bufb1, sem0, sem1,
                      half):
    wid, cid, sid = _worker_id()
    sidx = (sidx0, sidx1)
    didx = (didx0, didx1)
    bufa = (bufa0, bufa1)
    bufb = (bufb0, bufb1)
    sems = (sem0, sem1)

    def _rows(blk):
        rw = jnp.where(cid == FAST_CID, RW_FAST // 2, RW_SLOW // 2)
        base = jnp.where(cid == FAST_CID, sid * (RW_FAST // 2),
                         NS * (RW_FAST // 2) + sid * (RW_SLOW // 2))
        return base + blk * K_CLS, rw // K_CLS

    _, nblk = _rows(0)

    def fire(b, s):
        rowbase, _ = _rows(b)
        pltpu.sync_copy(src_hbm.at[pl.ds(rowbase, K_CLS)], sidx[s])
        pltpu.sync_copy(dst_hbm.at[pl.ds(rowbase, K_CLS)], didx[s])
        return [
            pltpu.async_copy(a_hbm.at[sidx[s].at[j]],
                             bufa[s].at[pl.ds(j * 128, 128)], sems[s])
            for j in range(K_CLS)
        ] + [
            pltpu.async_copy(b_hbm.at[didx[s].at[j]],
                             bufb[s].at[pl.ds(j * 128, 128)], sems[s])
            for j in range(K_CLS)
        ]

    def drain_consume(descs, s, b):
        rowbase, _ = _rows(b)
        ebase = rowbase * 128
        for d in descs:
            d.wait()
        pltpu.sync_copy(bufa[s],
                        o_hbm.at[pl.ds(ebase, K_CLS * 128), pl.ds(0, H)])
        pltpu.sync_copy(bufb[s],
                        o_hbm.at[pl.ds(ebase, K_CLS * 128), pl.ds(H, H)])

    d0 = fire(0, 0)

    def body(b2, carry):
        b = b2 * 2
        d1 = fire(b + 1, 1)
        drain_consume(d0, 0, b)

        @pl.when(b + 2 < nblk)
        def _():
            fire(b + 2, 0)

        drain_consume(d1, 1, b + 1)
        return carry

    lax.fori_loop(0, nblk // 2, body, 0)


def _make_edge_gather(half):
    return functools.partial(
        pl.kernel,
        out_type=jax.ShapeDtypeStruct((EP // 2, 2 * H), F32),
        mesh=_mesh(),
        compiler_params=pltpu.CompilerParams(use_tc_tiling_on_sc=False),
        scratch_types=[
            pltpu.VMEM((K_CLS, 128), jnp.int32),
            pltpu.VMEM((K_CLS, 128), jnp.int32),
            pltpu.VMEM((K_CLS, 128), jnp.int32),
            pltpu.VMEM((K_CLS, 128), jnp.int32),
            pltpu.VMEM((K_CLS * 128, H), F32),
            pltpu.VMEM((K_CLS * 128, H), F32),
            pltpu.VMEM((K_CLS * 128, H), F32),
            pltpu.VMEM((K_CLS * 128, H), F32),
            pltpu.SemaphoreType.DMA,
            pltpu.SemaphoreType.DMA,
        ],
    )(lambda *args: _edge_gather_body(*args, half=half))


_edge_gather_0 = _make_edge_gather(0)
_edge_gather_1 = _make_edge_gather(1)


def _tc1_body(deg0_ref, deg1_ref, x_ref, w_ref, dinv_ref, q_ref):
    deg = deg0_ref[...] + deg1_ref[...] + 1.0
    dinv = lax.rsqrt(deg)
    dinv_ref[...] = dinv
    q_ref[...] = jnp.dot(x_ref[...], w_ref[...],
                         preferred_element_type=F32) * dinv


_tc1 = pl.pallas_call(
    _tc1_body,
    out_shape=(
        jax.ShapeDtypeStruct((NP, 1), F32),
        jax.ShapeDtypeStruct((NP, H), F32),
    ),
)


def _tc2_body(s0_ref, s1_ref, q1_ref, dinv_ref, w2_ref, b1_ref, q2_ref):
    dinv = dinv_ref[...]
    h1 = jnp.maximum(
        dinv * (s0_ref[...] + s1_ref[...] + q1_ref[...]) + b1_ref[...], 0.0)
    q2_ref[...] = jnp.dot(h1, w2_ref[...], preferred_element_type=F32) * dinv


_tc2 = pl.pallas_call(
    _tc2_body,
    out_shape=jax.ShapeDtypeStruct((NP, H), F32),
)


def _tc3_body(s0_ref, s1_ref, q2_ref, dinv_ref, wc1_ref, b2_ref, bc1_ref,
              a_ref, b_ref):
    h2 = dinv_ref[...] * (s0_ref[...] + s1_ref[...] + q2_ref[...]) + b2_ref[...]
    wc1 = wc1_ref[...]
    a_ref[...] = jnp.dot(h2, wc1[:H], preferred_element_type=F32) + bc1_ref[...]
    b_ref[...] = jnp.dot(h2, wc1[H:], preferred_element_type=F32)


_tc3 = pl.pallas_call(
    _tc3_body,
    out_shape=(
        jax.ShapeDtypeStruct((NP, H), F32),
        jax.ShapeDtypeStruct((NP, H), F32),
    ),
)

_BR = 5120


def _tc4_body(z_ref, w_ref, bias_ref, o_ref):
    zc = z_ref[...]
    z = jnp.maximum(zc[:, :H] + zc[:, H:], 0.0)
    o_ref[...] = jax.lax.dot_general(
        w_ref[...], z, (((0,), (1,)), ((), ())),
        preferred_element_type=F32) + bias_ref[...]


_tc4 = pl.pallas_call(
    _tc4_body,
    grid=(EP // 2 // _BR,),
    in_specs=[
        pl.BlockSpec((_BR, 2 * H), lambda i: (i, 0)),
        pl.BlockSpec((H, 2), lambda i: (0, 0)),
        pl.BlockSpec((2, 1), lambda i: (0, 0)),
    ],
    out_specs=pl.BlockSpec((2, _BR), lambda i: (0, i)),
    out_shape=jax.ShapeDtypeStruct((2, EP // 2), F32),
)


def kernel(x, edge_index, W1, b1, W2, b2, Wc1, bc1, Wc2, bc2):
    src = edge_index[0]
    dst = edge_index[1]
    pad = jnp.full((EP - E,), N, dtype=jnp.int32)
    src2d = jnp.concatenate([src, pad]).reshape(ER, 128)
    dst2d = jnp.concatenate([dst, pad]).reshape(ER, 128)
    x_pad = jnp.pad(x, ((0, NP - N), (0, 0)))
    ones_deg = jnp.ones((128, DW), F32)
    zeros1 = jnp.zeros((NPT, DW), F32)
    zeros64 = jnp.zeros((NPT, H), F32)

    deg_p = _deg_kernel(dst2d, ones_deg, zeros1)
    dinv, q1 = _tc1(deg_p[0, :, :1], deg_p[1, :, :1], x_pad, W1)
    s1 = _agg_kernel(q1, src2d, dst2d, zeros64)
    q2 = _tc2(s1[0], s1[1], q1, dinv, W2, b1.reshape(1, H))
    s2 = _agg_kernel(q2, src2d, dst2d, zeros64)
    a_nodes, b_nodes = _tc3(s2[0], s2[1], q2, dinv, Wc1, b2.reshape(1, H),
                            bc1.reshape(1, H))
    hr = ER // 2
    bc2c = bc2.reshape(2, 1)
    zcat0 = _edge_gather_0(a_nodes, b_nodes, src2d[:hr], dst2d[:hr])
    zcat1 = _edge_gather_1(a_nodes, b_nodes, src2d[hr:], dst2d[hr:])
    out0 = _tc4(zcat0, Wc2, bc2c)
    out1 = _tc4(zcat1, Wc2, bc2c)
    out_t = jnp.concatenate([out0, out1], axis=1)
    return out_t[:, :E].T


# speedup vs baseline: 1.0671x; 1.0671x over previous
import functools

import jax
import jax.numpy as jnp
from jax import lax
from jax.experimental import pallas as pl
from jax.experimental.pallas import tpu as pltpu
from jax.experimental.pallas import tpu_sc as plsc

N = 10000
E = 320000
D_IN = 128
H = 64
F32 = jnp.float32

NC = 2
NS = 16
NW = NC * NS
NP = 10240
NPT = NP // NS
EP = 327680
ER = EP // 128
RW = ER // NW

K_AGG = 4
K_CLS = 2


def _mesh():
    return plsc.VectorSubcoreMesh(
        core_axis_name="c", subcore_axis_name="s", num_cores=NC, num_subcores=NS
    )


def _worker_id():
    return lax.axis_index("s") * NC + lax.axis_index("c"), lax.axis_index("c"), \
        lax.axis_index("s")


RW_FAST = 120
RW_SLOW = 40
FAST_CID = 0


def _edge_rows(cid, sid, blk, k):
    rw = jnp.where(cid == FAST_CID, RW_FAST, RW_SLOW)
    base = jnp.where(cid == FAST_CID, sid * RW_FAST,
                     NS * RW_FAST + sid * RW_SLOW)
    return base + blk * k, rw // k


DW = 16


@functools.partial(
    pl.kernel,
    out_type=jax.ShapeDtypeStruct((NC, NP, DW), F32),
    mesh=_mesh(),
    compiler_params=pltpu.CompilerParams(use_tc_tiling_on_sc=False),
    scratch_types=[
        pltpu.VMEM((K_AGG, 128), jnp.int32),
        pltpu.VMEM((128, DW), F32),
        pltpu.VMEM((NPT, DW), F32),
        pltpu.VMEM_SHARED((NP, DW), F32),
    ],
)
def _deg_kernel(dst_hbm, ones_hbm, zeros_hbm, out_hbm, idx_v, ones_v, stage_v,
                acc_sh):
    wid, cid, sid = _worker_id()
    pltpu.sync_copy(zeros_hbm, stage_v)
    pltpu.sync_copy(stage_v, acc_sh.at[pl.ds(sid * NPT, NPT)])
    pltpu.sync_copy(ones_hbm, ones_v)
    plsc.subcore_barrier()

    def body(b, carry):
        rowbase = wid * RW + b * K_AGG
        pltpu.sync_copy(dst_hbm.at[pl.ds(rowbase, K_AGG)], idx_v)
        for j in range(K_AGG):
            pltpu.sync_copy(ones_v, acc_sh.at[idx_v.at[j]], add=True)
        return carry

    lax.fori_loop(0, RW // K_AGG, body, 0)
    plsc.subcore_barrier()
    pltpu.sync_copy(acc_sh.at[pl.ds(sid * NPT, NPT)], stage_v)
    pltpu.sync_copy(stage_v, out_hbm.at[cid, pl.ds(sid * NPT, NPT)])


@functools.partial(
    pl.kernel,
    out_type=jax.ShapeDtypeStruct((NC, NP, H), F32),
    mesh=_mesh(),
    compiler_params=pltpu.CompilerParams(use_tc_tiling_on_sc=False),
    scratch_types=[
        pltpu.VMEM((K_AGG, 128), jnp.int32),
        pltpu.VMEM((K_AGG, 128), jnp.int32),
        pltpu.VMEM((K_AGG, 128), jnp.int32),
        pltpu.VMEM((K_AGG, 128), jnp.int32),
        pltpu.VMEM((K_AGG * 128, H), F32),
        pltpu.VMEM((K_AGG * 128, H), F32),
        pltpu.SemaphoreType.DMA,
        pltpu.SemaphoreType.DMA,
        pltpu.VMEM_SHARED((NP, H), F32),
    ],
)
def _agg_kernel(q_hbm, src_hbm, dst_hbm, zeros_hbm, out_hbm, sidx0, sidx1,
                didx0, didx1, rows0, rows1, sem0, sem1, acc_sh):
    wid, cid, sid = _worker_id()
    sidx = (sidx0, sidx1)
    didx = (didx0, didx1)
    rows = (rows0, rows1)
    sems = (sem0, sem1)
    _r0 = K_AGG * 128
    _r1 = NPT - _r0
    pltpu.sync_copy(zeros_hbm.at[pl.ds(0, _r0)], rows0)
    pltpu.sync_copy(zeros_hbm.at[pl.ds(_r0, _r1)], rows1.at[pl.ds(0, _r1)])
    pltpu.sync_copy(rows0, acc_sh.at[pl.ds(sid * NPT, _r0)])
    pltpu.sync_copy(rows1.at[pl.ds(0, _r1)],
                    acc_sh.at[pl.ds(sid * NPT + _r0, _r1)])
    plsc.subcore_barrier()
    _, nblk = _edge_rows(cid, sid, 0, K_AGG)

    def fire(b, s):
        rowbase, _ = _edge_rows(cid, sid, b, K_AGG)
        pltpu.sync_copy(src_hbm.at[pl.ds(rowbase, K_AGG)], sidx[s])
        pltpu.sync_copy(dst_hbm.at[pl.ds(rowbase, K_AGG)], didx[s])
        return [
            pltpu.async_copy(q_hbm.at[sidx[s].at[j]],
                             rows[s].at[pl.ds(j * 128, 128)], sems[s])
            for j in range(K_AGG)
        ]

    def drain_consume(descs, s):
        for d in descs:
            d.wait()
        for j in range(K_AGG):
            pltpu.sync_copy(rows[s].at[pl.ds(j * 128, 128)],
                            acc_sh.at[didx[s].at[j]], add=True)

    d0 = fire(0, 0)

    def body(b2, carry):
        b = b2 * 2
        d1 = fire(b + 1, 1)
        drain_consume(d0, 0)

        @pl.when(b + 2 < nblk)
        def _():
            fire(b + 2, 0)

        drain_consume(d1, 1)
        return carry

    lax.fori_loop(0, nblk // 2, body, 0)
    plsc.subcore_barrier()
    pltpu.sync_copy(acc_sh.at[pl.ds(sid * NPT, _r0)], rows0)
    pltpu.sync_copy(acc_sh.at[pl.ds(sid * NPT + _r0, _r1)],
                    rows1.at[pl.ds(0, _r1)])
    pltpu.sync_copy(rows0, out_hbm.at[cid, pl.ds(sid * NPT, _r0)])
    pltpu.sync_copy(rows1.at[pl.ds(0, _r1)],
                    out_hbm.at[cid, pl.ds(sid * NPT + _r0, _r1)])


@functools.partial(
    pl.kernel,
    out_type=jax.ShapeDtypeStruct((EP, 2 * H), F32),
    mesh=_mesh(),
    compiler_params=pltpu.CompilerParams(use_tc_tiling_on_sc=False),
    scratch_types=[
        pltpu.VMEM((K_CLS, 128), jnp.int32),
        pltpu.VMEM((K_CLS, 128), jnp.int32),
        pltpu.VMEM((K_CLS, 128), jnp.int32),
        pltpu.VMEM((K_CLS, 128), jnp.int32),
        pltpu.VMEM((K_CLS * 128, H), F32),
        pltpu.VMEM((K_CLS * 128, H), F32),
        pltpu.VMEM((K_CLS * 128, H), F32),
        pltpu.VMEM((K_CLS * 128, H), F32),
        pltpu.SemaphoreType.DMA,
        pltpu.SemaphoreType.DMA,
    ],
)
def _edge_gather_kernel(a_hbm, b_hbm, src_hbm, dst_hbm, o_hbm, sidx0, sidx1,
                        didx0, didx1, bufa0, bufa1, bufb0, bufb1, sem0, sem1):
    wid, cid, sid = _worker_id()
    sidx = (sidx0, sidx1)
    didx = (didx0, didx1)
    bufa = (bufa0, bufa1)
    bufb = (bufb0, bufb1)
    sems = (sem0, sem1)
    _, nblk = _edge_rows(cid, sid, 0, K_CLS)

    def fire(b, s):
        rowbase, _ = _edge_rows(cid, sid, b, K_CLS)
        pltpu.sync_copy(src_hbm.at[pl.ds(rowbase, K_CLS)], sidx[s])
        pltpu.sync_copy(dst_hbm.at[pl.ds(rowbase, K_CLS)], didx[s])
        return [
            pltpu.async_copy(a_hbm.at[sidx[s].at[j]],
                             bufa[s].at[pl.ds(j * 128, 128)], sems[s])
            for j in range(K_CLS)
        ] + [
            pltpu.async_copy(b_hbm.at[didx[s].at[j]],
                             bufb[s].at[pl.ds(j * 128, 128)], sems[s])
            for j in range(K_CLS)
        ]

    def drain_consume(descs, s, b):
        rowbase, _ = _edge_rows(cid, sid, b, K_CLS)
        ebase = rowbase * 128
        for d in descs:
            d.wait()
        pltpu.sync_copy(bufa[s],
                        o_hbm.at[pl.ds(ebase, K_CLS * 128), pl.ds(0, H)])
        pltpu.sync_copy(bufb[s],
                        o_hbm.at[pl.ds(ebase, K_CLS * 128), pl.ds(H, H)])

    d0 = fire(0, 0)

    def body(b2, carry):
        b = b2 * 2
        d1 = fire(b + 1, 1)
        drain_consume(d0, 0, b)

        @pl.when(b + 2 < nblk)
        def _():
            fire(b + 2, 0)

        drain_consume(d1, 1, b + 1)
        return carry

    lax.fori_loop(0, nblk // 2, body, 0)


def _tc1_body(deg0_ref, deg1_ref, x_ref, w_ref, dinv_ref, q_ref):
    deg = deg0_ref[...] + deg1_ref[...] + 1.0
    dinv = lax.rsqrt(deg)
    dinv_ref[...] = dinv
    q_ref[...] = jnp.dot(x_ref[...], w_ref[...],
                         preferred_element_type=F32) * dinv


_tc1 = pl.pallas_call(
    _tc1_body,
    out_shape=(
        jax.ShapeDtypeStruct((NP, 1), F32),
        jax.ShapeDtypeStruct((NP, H), F32),
    ),
)


def _tc2_body(s0_ref, s1_ref, q1_ref, dinv_ref, w2_ref, b1_ref, q2_ref):
    dinv = dinv_ref[...]
    h1 = jnp.maximum(
        dinv * (s0_ref[...] + s1_ref[...] + q1_ref[...]) + b1_ref[...], 0.0)
    q2_ref[...] = jnp.dot(h1, w2_ref[...], preferred_element_type=F32) * dinv


_tc2 = pl.pallas_call(
    _tc2_body,
    out_shape=jax.ShapeDtypeStruct((NP, H), F32),
)


def _tc3_body(s0_ref, s1_ref, q2_ref, dinv_ref, wc1_ref, b2_ref, bc1_ref,
              a_ref, b_ref):
    h2 = dinv_ref[...] * (s0_ref[...] + s1_ref[...] + q2_ref[...]) + b2_ref[...]
    wc1 = wc1_ref[...]
    a_ref[...] = jnp.dot(h2, wc1[:H], preferred_element_type=F32) + bc1_ref[...]
    b_ref[...] = jnp.dot(h2, wc1[H:], preferred_element_type=F32)


_tc3 = pl.pallas_call(
    _tc3_body,
    out_shape=(
        jax.ShapeDtypeStruct((NP, H), F32),
        jax.ShapeDtypeStruct((NP, H), F32),
    ),
)

_BR = 5120


def _tc4_body(z_ref, w_ref, bias_ref, o_ref):
    zc = z_ref[...]
    z = jnp.maximum(zc[:, :H] + zc[:, H:], 0.0)
    o_ref[...] = jax.lax.dot_general(
        w_ref[...], z, (((0,), (1,)), ((), ())),
        preferred_element_type=F32) + bias_ref[...]


_tc4 = pl.pallas_call(
    _tc4_body,
    grid=(EP // _BR,),
    in_specs=[
        pl.BlockSpec((_BR, 2 * H), lambda i: (i, 0)),
        pl.BlockSpec((H, 2), lambda i: (0, 0)),
        pl.BlockSpec((2, 1), lambda i: (0, 0)),
    ],
    out_specs=pl.BlockSpec((2, _BR), lambda i: (0, i)),
    out_shape=jax.ShapeDtypeStruct((2, EP), F32),
)


def kernel(x, edge_index, W1, b1, W2, b2, Wc1, bc1, Wc2, bc2):
    src = edge_index[0]
    dst = edge_index[1]
    pad = jnp.full((EP - E,), N, dtype=jnp.int32)
    src2d = jnp.concatenate([src, pad]).reshape(ER, 128)
    dst2d = jnp.concatenate([dst, pad]).reshape(ER, 128)
    x_pad = jnp.pad(x, ((0, NP - N), (0, 0)))
    ones_deg = jnp.ones((128, DW), F32)
    zeros1 = jnp.zeros((NPT, DW), F32)
    zeros64 = jnp.zeros((NPT, H), F32)

    deg_p = _deg_kernel(dst2d, ones_deg, zeros1)
    dinv, q1 = _tc1(deg_p[0, :, :1], deg_p[1, :, :1], x_pad, W1)
    s1 = _agg_kernel(q1, src2d, dst2d, zeros64)
    q2 = _tc2(s1[0], s1[1], q1, dinv, W2, b1.reshape(1, H))
    s2 = _agg_kernel(q2, src2d, dst2d, zeros64)
    a_nodes, b_nodes = _tc3(s2[0], s2[1], q2, dinv, Wc1, b2.reshape(1, H),
                            bc1.reshape(1, H))
    zcat = _edge_gather_kernel(a_nodes, b_nodes, src2d, dst2d)
    out_t = _tc4(zcat, Wc2, bc2.reshape(2, 1))
    return out_t[:, :E].T
